# baseline (speedup 1.0000x reference)
import functools

import jax
import jax.numpy as jnp
from jax.experimental import pallas as pl

TEMP = 0.07
BLOCK_Q = 32


def _dist_block(a_ref, b_ref, o_ref):
    cross = jax.lax.dot_general(
        a_ref[...], b_ref[...], (((1,), (0,)), ((), ())),
        preferred_element_type=jnp.float32)
    o_ref[...] = jnp.minimum(cross, 0.0)


@jax.jit
def kernel(inputs, context):
    q, dim = inputs.shape
    k = context.shape[0]
    x_sq = jnp.sum(inputs * inputs, axis=1, keepdims=True)
    y_sq = jnp.sum(context * context, axis=1)[None, :]
    a = jnp.concatenate(
        [inputs * (2.0 / TEMP), -x_sq / TEMP, jnp.ones((q, 1), jnp.float32)],
        axis=1)
    b = jnp.concatenate(
        [context.T, jnp.ones((1, k), jnp.float32), -y_sq / TEMP],
        axis=0)
    grid = (q // BLOCK_Q,)
    return pl.pallas_call(
        _dist_block,
        grid=grid,
        in_specs=[
            pl.BlockSpec((BLOCK_Q, dim + 2), lambda i: (i, 0)),
            pl.BlockSpec((dim + 2, k), lambda i: (0, 0)),
        ],
        out_specs=pl.BlockSpec((BLOCK_Q, k), lambda i: (i, 0)),
        out_shape=jax.ShapeDtypeStruct((q, k), jnp.float32),
    )(a, b)


# speedup vs baseline: 2.6967x; 2.6967x over previous
import functools

import jax
import jax.numpy as jnp
from jax.experimental import pallas as pl

TEMP = 0.07
BLOCK_K = 2048


def _dist_block(b_ref, a_ref, o_ref):
    cross = jax.lax.dot_general(
        b_ref[...], a_ref[...], (((1,), (0,)), ((), ())),
        preferred_element_type=jnp.float32)
    o_ref[...] = jnp.minimum(cross, 0.0)


@jax.jit
def kernel(inputs, context):
    q, dim = inputs.shape
    k = context.shape[0]
    x_sq = jnp.sum(inputs * inputs, axis=1)[None, :]
    y_sq = jnp.sum(context * context, axis=1, keepdims=True)
    b = jnp.concatenate(
        [context * (2.0 / TEMP), -y_sq / TEMP, jnp.ones((k, 1), jnp.float32)],
        axis=1)
    a = jnp.concatenate(
        [inputs.T, jnp.ones((1, q), jnp.float32), -x_sq / TEMP],
        axis=0)
    grid = (pl.cdiv(k, BLOCK_K),)
    out_t = pl.pallas_call(
        _dist_block,
        grid=grid,
        in_specs=[
            pl.BlockSpec((BLOCK_K, dim + 2), lambda i: (i, 0)),
            pl.BlockSpec((dim + 2, q), lambda i: (0, 0)),
        ],
        out_specs=pl.BlockSpec((BLOCK_K, q), lambda i: (i, 0)),
        out_shape=jax.ShapeDtypeStruct((k, q), jnp.float32),
    )(b, a)
    return out_t.T


# speedup vs baseline: 2.7476x; 1.0189x over previous
import functools

import jax
import jax.numpy as jnp
from jax.experimental import pallas as pl

TEMP = 0.07
BLOCK_K = 4096


def _dist_block(b_ref, a_ref, o_ref):
    cross = jax.lax.dot_general(
        b_ref[...], a_ref[...], (((1,), (0,)), ((), ())),
        preferred_element_type=jnp.float32)
    o_ref[...] = jnp.minimum(cross, 0.0)


@jax.jit
def kernel(inputs, context):
    q, dim = inputs.shape
    k = context.shape[0]
    x_sq = jnp.sum(inputs * inputs, axis=1)[None, :]
    y_sq = jnp.sum(context * context, axis=1, keepdims=True)
    b = jnp.concatenate(
        [context * (2.0 / TEMP), -y_sq / TEMP, jnp.ones((k, 1), jnp.float32)],
        axis=1)
    a = jnp.concatenate(
        [inputs.T, jnp.ones((1, q), jnp.float32), -x_sq / TEMP],
        axis=0)
    grid = (pl.cdiv(k, BLOCK_K),)
    out_t = pl.pallas_call(
        _dist_block,
        grid=grid,
        in_specs=[
            pl.BlockSpec((BLOCK_K, dim + 2), lambda i: (i, 0)),
            pl.BlockSpec((dim + 2, q), lambda i: (0, 0)),
        ],
        out_specs=pl.BlockSpec((BLOCK_K, q), lambda i: (i, 0)),
        out_shape=jax.ShapeDtypeStruct((k, q), jnp.float32),
    )(b, a)
    return out_t.T


# speedup vs baseline: 3.5399x; 1.2884x over previous
import functools

import jax
import jax.numpy as jnp
from jax.experimental import pallas as pl

TEMP = 0.07
BLOCK_K = 4096


def _dist_block(b_ref, a_ref, o_ref):
    cross = jax.lax.dot_general(
        b_ref[...], a_ref[...], (((0,), (0,)), ((), ())),
        preferred_element_type=jnp.float32)
    o_ref[...] = jnp.minimum(cross, 0.0)


@jax.jit
def kernel(inputs, context):
    q, dim = inputs.shape
    k = context.shape[0]
    x_sq = jnp.sum(inputs * inputs, axis=1)[None, :]
    y_sq = jnp.sum(context * context, axis=1)[None, :]
    bt = jnp.concatenate(
        [context.T * (2.0 / TEMP), -y_sq / TEMP, jnp.ones((1, k), jnp.float32)],
        axis=0)
    a = jnp.concatenate(
        [inputs.T, jnp.ones((1, q), jnp.float32), -x_sq / TEMP],
        axis=0)
    grid = (pl.cdiv(k, BLOCK_K),)
    out_t = pl.pallas_call(
        _dist_block,
        grid=grid,
        in_specs=[
            pl.BlockSpec((dim + 2, BLOCK_K), lambda i: (0, i)),
            pl.BlockSpec((dim + 2, q), lambda i: (0, 0)),
        ],
        out_specs=pl.BlockSpec((BLOCK_K, q), lambda i: (i, 0)),
        out_shape=jax.ShapeDtypeStruct((k, q), jnp.float32),
    )(bt, a)
    return out_t.T
